# baseline (speedup 1.0000x reference)
import functools

import jax
import jax.numpy as jnp
from jax import lax
from jax.experimental import pallas as pl
from jax.experimental.pallas import tpu as pltpu
from jax.experimental.pallas import tpu_sc as plsc

_SIZE = 100000
_DIM = 128
_BATCH = 16384
_NCOL = 50
_NPAD = 64
_EPS = 1e-5
_BOUNDARY = 1.0 - _EPS

_NC = 2
_NS = 16
_NW = _NC * _NS
_ROWS_PER_W = _BATCH // _NW
_R = 8
_NCH = _ROWS_PER_W // _R
_GROWS = 100
_NG = _R * _NCOL // _GROWS

_GDN = lax.GatherDimensionNumbers(
    offset_dims=(), collapsed_slice_dims=(0,), start_index_map=(0,))


def _vperm(vec, idx):
    return lax.gather(vec, idx[:, None], dimension_numbers=_GDN,
                      slice_sizes=(1,),
                      mode=lax.GatherScatterMode.PROMISE_IN_BOUNDS)


def _sc_body(idx_hbm, w_hbm, rnorm_hbm, nrm_hbm, dot_hbm,
             idx_a, idx_b,
             ra0, ra1, ra2, ra3, rb0, rb1, rb2, rb3,
             na_v, nb_v, nrm_v, dot_v,
             sa0, sa1, sa2, sa3, sb0, sb1, sb2, sb3):
    rows_a = (ra0, ra1, ra2, ra3)
    rows_b = (rb0, rb1, rb2, rb3)
    sems_a = (sa0, sa1, sa2, sa3)
    sems_b = (sb0, sb1, sb2, sb3)
    cid = lax.axis_index("c")
    sid = lax.axis_index("s")
    wid = cid * _NS + sid
    iota16 = lax.iota(jnp.int32, 16)
    rowids = [jnp.minimum(g * 16 + iota16, _NCOL - 1) for g in range(4)]
    rots = [jnp.bitwise_and(iota16 + j2, 15) for j2 in range(16)]

    def stage_idx(c, idx_v):
        i0 = wid * (_ROWS_PER_W * _NCOL // _GROWS) + c * _NG
        pltpu.sync_copy(idx_hbm.at[pl.ds(i0, _NG)], idx_v)

    def fire(idx_v, j, rows_v, nbuf, sems):
        pltpu.async_copy(w_hbm.at[idx_v.at[j]], rows_v[j], sems[j])
        pltpu.async_copy(rnorm_hbm.at[idx_v.at[j]], nbuf.at[j], sems[j])

    def drain(idx_v, j, rows_v, nbuf, sems):
        pltpu.make_async_copy(
            w_hbm.at[idx_v.at[j]], rows_v[j], sems[j]).wait()
        pltpu.make_async_copy(
            rnorm_hbm.at[idx_v.at[j]], nbuf.at[j], sems[j]).wait()

    def compute2(j, rref, nbuf):
        jspl = jnp.full((16,), j, jnp.int32)
        for r in (2 * j, 2 * j + 1):
            base = (r % 2) * _NCOL
            rids = [base + rowids[g] for g in range(4)]

            def dbody(dc, accs, rref=rref, rids=rids, base=base):
                new = list(accs)
                ucv = rref[base, pl.ds(dc * 16, 16)]
                for j2 in range(16):
                    dswz = dc * 16 + rots[j2]
                    u_g = _vperm(ucv, rots[j2])
                    for g in range(4):
                        vg = plsc.load_gather(rref, [rids[g], dswz])
                        new[g] = new[g] + vg * u_g
                return tuple(new)

            zero = jnp.zeros((16,), jnp.float32)
            accs = lax.fori_loop(0, _DIM // 16, dbody, (zero,) * 4)
            for g in range(4):
                dot_v[pl.ds(r * _NPAD + g * 16, 16)] = accs[g]
                nv = plsc.load_gather(nbuf, [jspl, rids[g]])
                nrm_v[pl.ds(r * _NPAD + g * 16, 16)] = nv

    def flush(c):
        r0 = wid * _ROWS_PER_W + c * _R
        pltpu.sync_copy(nrm_v, nrm_hbm.at[pl.ds(r0 * _NPAD, _R * _NPAD)])
        pltpu.sync_copy(dot_v, dot_hbm.at[pl.ds(r0 * _NPAD, _R * _NPAD)])

    stage_idx(0, idx_a)
    for j in range(_NG):
        fire(idx_a, j, rows_a, na_v, sems_a)

    def outer(i, carry):
        c0 = i * 2
        stage_idx(c0 + 1, idx_b)
        for j in range(_NG):
            drain(idx_a, j, rows_a, na_v, sems_a)
            compute2(j, rows_a[j], na_v)
            fire(idx_b, j, rows_b, nb_v, sems_b)
        flush(c0)
        @pl.when(i + 1 < _NCH // 2)
        def _():
            stage_idx(c0 + 2, idx_a)

        for j in range(_NG):
            drain(idx_b, j, rows_b, nb_v, sems_b)
            compute2(j, rows_b[j], nb_v)

            @pl.when(i + 1 < _NCH // 2)
            def _(j=j):
                fire(idx_a, j, rows_a, na_v, sems_a)

        flush(c0 + 1)
        return carry

    lax.fori_loop(0, _NCH // 2, outer, 0)


def _tc_norms_body(w_ref, out_ref):
    w = w_ref[...]
    out_ref[...] = jnp.sum(w * w, axis=-1)


def _tc_body(nrm_ref, dot_ref, out_ref):
    nrm = nrm_ref[...]
    dot = dot_ref[...]
    a = nrm[:, 0:1]
    d0 = dot[:, 0:1]
    sa = jnp.sqrt(a)
    su = jnp.where(sa > 1.0, 1.0 / jnp.maximum(sa, _EPS), 1.0)
    sc = jnp.sqrt(nrm)
    sv = jnp.where(sc > 1.0, 1.0 / jnp.maximum(sc, _EPS), 1.0)
    squ = jnp.clip(su * su * a, 0.0, _BOUNDARY)
    sqv = jnp.clip(sv * sv * nrm, 0.0, _BOUNDARY)
    sqd = su * su * a + sv * sv * nrm - 2.0 * (su * sv) * dot
    x = sqd / ((1.0 - squ) * (1.0 - sqv)) * 2.0 + 1.0
    z = jnp.sqrt(jnp.maximum(x * x - 1.0, 0.0))
    del d0
    out_ref[...] = jnp.log(x + z)


@jax.jit
def kernel(inputs, weight):
    idx = inputs.reshape(_BATCH * _NCOL // _GROWS, _GROWS)

    w4 = weight.reshape(10, 10, 1000, _DIM)
    rnorm = pl.pallas_call(
        _tc_norms_body,
        grid=(10,),
        in_specs=[pl.BlockSpec((1, 10, 1000, _DIM), lambda i: (i, 0, 0, 0))],
        out_specs=pl.BlockSpec((1, 10, 1000), lambda i: (i, 0, 0)),
        out_shape=jax.ShapeDtypeStruct((10, 10, 1000), jnp.float32),
    )(w4)
    rnorm = rnorm.reshape(_SIZE)

    sc_call = pl.kernel(
        _sc_body,
        out_type=(
            jax.ShapeDtypeStruct((_BATCH * _NPAD,), jnp.float32),
            jax.ShapeDtypeStruct((_BATCH * _NPAD,), jnp.float32),
        ),
        mesh=plsc.VectorSubcoreMesh(
            core_axis_name="c", subcore_axis_name="s",
            num_cores=_NC, num_subcores=_NS),
        compiler_params=pltpu.CompilerParams(needs_layout_passes=False),
        scratch_types=(
            [pltpu.VMEM((_NG, _GROWS), jnp.int32)] * 2
            + [pltpu.VMEM((_GROWS, _DIM), jnp.float32)] * (2 * _NG)
            + [pltpu.VMEM((_NG, _GROWS), jnp.float32)] * 2
            + [pltpu.VMEM((_R * _NPAD,), jnp.float32)] * 2
            + [pltpu.SemaphoreType.DMA] * (2 * _NG)
        ),
    )
    nrm, dot = sc_call(idx, weight, rnorm)
    nrm = nrm.reshape(_BATCH, _NPAD)
    dot = dot.reshape(_BATCH, _NPAD)

    blk = 1024
    full = pl.pallas_call(
        _tc_body,
        grid=(_BATCH // blk,),
        in_specs=[
            pl.BlockSpec((blk, _NPAD), lambda i: (i, 0)),
            pl.BlockSpec((blk, _NPAD), lambda i: (i, 0)),
        ],
        out_specs=pl.BlockSpec((blk, _NPAD), lambda i: (i, 0)),
        out_shape=jax.ShapeDtypeStruct((_BATCH, _NPAD), jnp.float32),
    )(nrm, dot)
    return full[:, 1:_NCOL]


# speedup vs baseline: 1.5964x; 1.5964x over previous
import functools

import jax
import jax.numpy as jnp
from jax import lax
from jax.experimental import pallas as pl
from jax.experimental.pallas import tpu as pltpu
from jax.experimental.pallas import tpu_sc as plsc

_SIZE = 100000
_DIM = 128
_BATCH = 16384
_NCOL = 50
_NPAD = 64
_EPS = 1e-5
_BOUNDARY = 1.0 - _EPS

_NC = 2
_NS = 16
_NW = _NC * _NS
_ROWS_PER_W = _BATCH // _NW
_R = 8
_NCH = _ROWS_PER_W // _R
_GROWS = 100
_NG = _R * _NCOL // _GROWS

_GDN = lax.GatherDimensionNumbers(
    offset_dims=(), collapsed_slice_dims=(0,), start_index_map=(0,))


def _vperm(vec, idx):
    return lax.gather(vec, idx[:, None], dimension_numbers=_GDN,
                      slice_sizes=(1,),
                      mode=lax.GatherScatterMode.PROMISE_IN_BOUNDS)


def _sc_body(idx_hbm, w_hbm, out_hbm,
             idx_a, idx_b,
             ra0, ra1, ra2, ra3, rb0, rb1, rb2, rb3,
             out_a, out_b,
             sa0, sa1, sa2, sa3, sb0, sb1, sb2, sb3, osem_a, osem_b):
    rows_a = (ra0, ra1, ra2, ra3)
    rows_b = (rb0, rb1, rb2, rb3)
    sems_a = (sa0, sa1, sa2, sa3)
    sems_b = (sb0, sb1, sb2, sb3)
    cid = lax.axis_index("c")
    sid = lax.axis_index("s")
    wid = cid * _NS + sid
    iota16 = lax.iota(jnp.int32, 16)
    rowids = [jnp.minimum(g * 16 + iota16, _NCOL - 1) for g in range(4)]
    rots = [jnp.bitwise_and(iota16 + j2, 15) for j2 in range(16)]

    def stage_idx(c, idx_v):
        i0 = wid * (_ROWS_PER_W * _NCOL // _GROWS) + c * _NG
        pltpu.sync_copy(idx_hbm.at[pl.ds(i0, _NG)], idx_v)

    def fire(idx_v, j, rows_v, sems):
        pltpu.async_copy(w_hbm.at[idx_v.at[j]], rows_v[j], sems[j])

    def drain(idx_v, j, rows_v, sems):
        pltpu.make_async_copy(
            w_hbm.at[idx_v.at[j]], rows_v[j], sems[j]).wait()

    def compute2(j, rref, out_v):
        for r in (2 * j, 2 * j + 1):
            base = (r % 2) * _NCOL
            rids = [base + rowids[g] for g in range(4)]

            def dbody(dc, accs, rref=rref, rids=rids, base=base):
                new = list(accs)
                ucv = rref[base, pl.ds(dc * 16, 16)]
                for j2 in range(16):
                    dswz = dc * 16 + rots[j2]
                    u_g = _vperm(ucv, rots[j2])
                    for g in range(4):
                        vg = plsc.load_gather(rref, [rids[g], dswz])
                        new[2 * g] = new[2 * g] + vg * u_g
                        new[2 * g + 1] = new[2 * g + 1] + vg * vg
                return tuple(new)

            zero = jnp.zeros((16,), jnp.float32)
            accs = lax.fori_loop(0, _DIM // 16, dbody, (zero,) * 8)
            for g in range(4):
                out_v[pl.ds(r * 2 * _NPAD + g * 16, 16)] = accs[2 * g + 1]
                out_v[pl.ds(r * 2 * _NPAD + _NPAD + g * 16, 16)] = accs[2 * g]

    def flush(c, out_v, osem):
        r0 = wid * _ROWS_PER_W + c * _R
        pltpu.async_copy(
            out_v, out_hbm.at[pl.ds(r0 * 2 * _NPAD, _R * 2 * _NPAD)], osem)

    def flush_wait(c, out_v, osem):
        r0 = wid * _ROWS_PER_W + c * _R
        pltpu.make_async_copy(
            out_v, out_hbm.at[pl.ds(r0 * 2 * _NPAD, _R * 2 * _NPAD)],
            osem).wait()

    stage_idx(0, idx_a)
    for j in range(_NG):
        fire(idx_a, j, rows_a, sems_a)

    def outer(i, carry):
        c0 = i * 2
        stage_idx(c0 + 1, idx_b)

        @pl.when(i > 0)
        def _():
            flush_wait(c0 - 2, out_a, osem_a)

        for j in range(_NG):
            drain(idx_a, j, rows_a, sems_a)
            compute2(j, rows_a[j], out_a)
            fire(idx_b, j, rows_b, sems_b)
        flush(c0, out_a, osem_a)
        @pl.when(i + 1 < _NCH // 2)
        def _():
            stage_idx(c0 + 2, idx_a)

        @pl.when(i > 0)
        def _():
            flush_wait(c0 - 1, out_b, osem_b)

        for j in range(_NG):
            drain(idx_b, j, rows_b, sems_b)
            compute2(j, rows_b[j], out_b)

            @pl.when(i + 1 < _NCH // 2)
            def _(j=j):
                fire(idx_a, j, rows_a, sems_a)

        flush(c0 + 1, out_b, osem_b)
        return carry

    lax.fori_loop(0, _NCH // 2, outer, 0)
    flush_wait(_NCH - 2, out_a, osem_a)
    flush_wait(_NCH - 1, out_b, osem_b)


def _tc_body(x_ref, out_ref):
    nrm = x_ref[:, :_NPAD]
    dot = x_ref[:, _NPAD:]
    a = nrm[:, 0:1]
    d0 = dot[:, 0:1]
    sa = jnp.sqrt(a)
    su = jnp.where(sa > 1.0, 1.0 / jnp.maximum(sa, _EPS), 1.0)
    sc = jnp.sqrt(nrm)
    sv = jnp.where(sc > 1.0, 1.0 / jnp.maximum(sc, _EPS), 1.0)
    squ = jnp.clip(su * su * a, 0.0, _BOUNDARY)
    sqv = jnp.clip(sv * sv * nrm, 0.0, _BOUNDARY)
    sqd = su * su * a + sv * sv * nrm - 2.0 * (su * sv) * dot
    x = sqd / ((1.0 - squ) * (1.0 - sqv)) * 2.0 + 1.0
    z = jnp.sqrt(jnp.maximum(x * x - 1.0, 0.0))
    del d0
    out_ref[...] = jnp.log(x + z)


@jax.jit
def kernel(inputs, weight):
    idx = inputs.reshape(_BATCH * _NCOL // _GROWS, _GROWS)

    sc_call = pl.kernel(
        _sc_body,
        out_type=(
            jax.ShapeDtypeStruct((_BATCH * 2 * _NPAD,), jnp.float32),
        ),
        mesh=plsc.VectorSubcoreMesh(
            core_axis_name="c", subcore_axis_name="s",
            num_cores=_NC, num_subcores=_NS),
        compiler_params=pltpu.CompilerParams(needs_layout_passes=False),
        scratch_types=(
            [pltpu.VMEM((_NG, _GROWS), jnp.int32)] * 2
            + [pltpu.VMEM((_GROWS, _DIM), jnp.float32)] * (2 * _NG)
            + [pltpu.VMEM((_R * 2 * _NPAD,), jnp.float32)] * 2
            + [pltpu.SemaphoreType.DMA] * (2 * _NG + 2)
        ),
    )
    (nd,) = sc_call(idx, weight)
    nd = nd.reshape(_BATCH, 2 * _NPAD)

    blk = 1024
    full = pl.pallas_call(
        _tc_body,
        grid=(_BATCH // blk,),
        in_specs=[
            pl.BlockSpec((blk, 2 * _NPAD), lambda i: (i, 0)),
        ],
        out_specs=pl.BlockSpec((blk, _NPAD), lambda i: (i, 0)),
        out_shape=jax.ShapeDtypeStruct((_BATCH, _NPAD), jnp.float32),
    )(nd)
    return full[:, 1:_NCOL]


# speedup vs baseline: 1.7195x; 1.0771x over previous
import functools

import jax
import jax.numpy as jnp
from jax import lax
from jax.experimental import pallas as pl
from jax.experimental.pallas import tpu as pltpu
from jax.experimental.pallas import tpu_sc as plsc

_SIZE = 100000
_DIM = 128
_BATCH = 16384
_NCOL = 50
_NPAD = 64
_EPS = 1e-5
_BOUNDARY = 1.0 - _EPS

_NC = 2
_NS = 16
_NW = _NC * _NS
_ROWS_PER_W = _BATCH // _NW
_R = 8
_NCH = _ROWS_PER_W // _R
_GROWS = 100
_NG = _R * _NCOL // _GROWS

_GDN = lax.GatherDimensionNumbers(
    offset_dims=(), collapsed_slice_dims=(0,), start_index_map=(0,))


def _vperm(vec, idx):
    return lax.gather(vec, idx[:, None], dimension_numbers=_GDN,
                      slice_sizes=(1,),
                      mode=lax.GatherScatterMode.PROMISE_IN_BOUNDS)


def _sc_body(idx_hbm, w_hbm, out_hbm,
             idx_a, idx_b,
             ra0, ra1, ra2, ra3, rb0, rb1, rb2, rb3,
             out_a, out_b,
             sa0, sa1, sa2, sa3, sb0, sb1, sb2, sb3, osem_a, osem_b,
             isem_a, isem_b):
    rows_a = (ra0, ra1, ra2, ra3)
    rows_b = (rb0, rb1, rb2, rb3)
    sems_a = (sa0, sa1, sa2, sa3)
    sems_b = (sb0, sb1, sb2, sb3)
    cid = lax.axis_index("c")
    sid = lax.axis_index("s")
    wid = cid * _NS + sid
    iota16 = lax.iota(jnp.int32, 16)
    rowids = [jnp.minimum(g * 16 + iota16, _NCOL - 1) for g in range(4)]
    rots = [jnp.bitwise_and(iota16 + j2, 15) for j2 in range(16)]

    def stage_idx(c, idx_v, isem):
        i0 = wid * (_ROWS_PER_W * _NCOL // _GROWS) + c * _NG
        pltpu.async_copy(idx_hbm.at[pl.ds(i0, _NG)], idx_v, isem)

    def stage_wait(c, idx_v, isem):
        i0 = wid * (_ROWS_PER_W * _NCOL // _GROWS) + c * _NG
        pltpu.make_async_copy(idx_hbm.at[pl.ds(i0, _NG)], idx_v, isem).wait()

    def fire(idx_v, j, rows_v, sems):
        pltpu.async_copy(w_hbm.at[idx_v.at[j]], rows_v[j], sems[j])

    def drain(idx_v, j, rows_v, sems):
        pltpu.make_async_copy(
            w_hbm.at[idx_v.at[j]], rows_v[j], sems[j]).wait()

    def compute2(j, rref, out_v):
        for r in (2 * j, 2 * j + 1):
            base = (r % 2) * _NCOL
            rids = [base + rowids[g] for g in range(4)]

            def dbody(dc, accs, rref=rref, rids=rids, base=base):
                new = list(accs)
                ucv = rref[base, pl.ds(dc * 16, 16)]
                for j2 in range(16):
                    dswz = dc * 16 + rots[j2]
                    u_g = _vperm(ucv, rots[j2])
                    for g in range(4):
                        vg = plsc.load_gather(rref, [rids[g], dswz])
                        new[2 * g] = new[2 * g] + vg * u_g
                        new[2 * g + 1] = new[2 * g + 1] + vg * vg
                return tuple(new)

            zero = jnp.zeros((16,), jnp.float32)
            accs = lax.fori_loop(0, _DIM // 16, dbody, (zero,) * 8)
            for g in range(4):
                out_v[pl.ds(r * 2 * _NPAD + g * 16, 16)] = accs[2 * g + 1]
                out_v[pl.ds(r * 2 * _NPAD + _NPAD + g * 16, 16)] = accs[2 * g]

    def flush(c, out_v, osem):
        r0 = wid * _ROWS_PER_W + c * _R
        pltpu.async_copy(
            out_v, out_hbm.at[pl.ds(r0 * 2 * _NPAD, _R * 2 * _NPAD)], osem)

    def flush_wait(c, out_v, osem):
        r0 = wid * _ROWS_PER_W + c * _R
        pltpu.make_async_copy(
            out_v, out_hbm.at[pl.ds(r0 * 2 * _NPAD, _R * 2 * _NPAD)],
            osem).wait()

    stage_idx(0, idx_a, isem_a)
    stage_wait(0, idx_a, isem_a)
    for j in range(_NG):
        fire(idx_a, j, rows_a, sems_a)

    def outer(i, carry):
        c0 = i * 2
        stage_idx(c0 + 1, idx_b, isem_b)

        @pl.when(i > 0)
        def _():
            flush_wait(c0 - 2, out_a, osem_a)

        for j in range(_NG):
            drain(idx_a, j, rows_a, sems_a)
            compute2(j, rows_a[j], out_a)
            if j == 0:
                stage_wait(c0 + 1, idx_b, isem_b)
            fire(idx_b, j, rows_b, sems_b)
        flush(c0, out_a, osem_a)
        @pl.when(i + 1 < _NCH // 2)
        def _():
            stage_idx(c0 + 2, idx_a, isem_a)

        @pl.when(i > 0)
        def _():
            flush_wait(c0 - 1, out_b, osem_b)

        for j in range(_NG):
            drain(idx_b, j, rows_b, sems_b)
            compute2(j, rows_b[j], out_b)

            @pl.when(i + 1 < _NCH // 2)
            def _(j=j):
                if j == 0:
                    stage_wait(c0 + 2, idx_a, isem_a)
                fire(idx_a, j, rows_a, sems_a)

        flush(c0 + 1, out_b, osem_b)
        return carry

    lax.fori_loop(0, _NCH // 2, outer, 0)
    flush_wait(_NCH - 2, out_a, osem_a)
    flush_wait(_NCH - 1, out_b, osem_b)


def _tc_body(x_ref, out_ref):
    nrm = x_ref[:, :_NPAD]
    dot = x_ref[:, _NPAD:]
    a = nrm[:, 0:1]
    d0 = dot[:, 0:1]
    sa = jnp.sqrt(a)
    su = jnp.where(sa > 1.0, 1.0 / jnp.maximum(sa, _EPS), 1.0)
    sc = jnp.sqrt(nrm)
    sv = jnp.where(sc > 1.0, 1.0 / jnp.maximum(sc, _EPS), 1.0)
    squ = jnp.clip(su * su * a, 0.0, _BOUNDARY)
    sqv = jnp.clip(sv * sv * nrm, 0.0, _BOUNDARY)
    sqd = su * su * a + sv * sv * nrm - 2.0 * (su * sv) * dot
    x = sqd / ((1.0 - squ) * (1.0 - sqv)) * 2.0 + 1.0
    z = jnp.sqrt(jnp.maximum(x * x - 1.0, 0.0))
    del d0
    out_ref[...] = jnp.log(x + z)


@jax.jit
def kernel(inputs, weight):
    idx = inputs.reshape(_BATCH * _NCOL // _GROWS, _GROWS)

    sc_call = pl.kernel(
        _sc_body,
        out_type=(
            jax.ShapeDtypeStruct((_BATCH * 2 * _NPAD,), jnp.float32),
        ),
        mesh=plsc.VectorSubcoreMesh(
            core_axis_name="c", subcore_axis_name="s",
            num_cores=_NC, num_subcores=_NS),
        compiler_params=pltpu.CompilerParams(needs_layout_passes=False),
        scratch_types=(
            [pltpu.VMEM((_NG, _GROWS), jnp.int32)] * 2
            + [pltpu.VMEM((_GROWS, _DIM), jnp.float32)] * (2 * _NG)
            + [pltpu.VMEM((_R * 2 * _NPAD,), jnp.float32)] * 2
            + [pltpu.SemaphoreType.DMA] * (2 * _NG + 4)
        ),
    )
    (nd,) = sc_call(idx, weight)
    nd = nd.reshape(_BATCH, 2 * _NPAD)

    blk = 1024
    full = pl.pallas_call(
        _tc_body,
        grid=(_BATCH // blk,),
        in_specs=[
            pl.BlockSpec((blk, 2 * _NPAD), lambda i: (i, 0)),
        ],
        out_specs=pl.BlockSpec((blk, _NPAD), lambda i: (i, 0)),
        out_shape=jax.ShapeDtypeStruct((_BATCH, _NPAD), jnp.float32),
    )(nd)
    return full[:, 1:_NCOL]
